# baseline (device time: 191501 ns/iter reference)
import functools

import jax
import jax.numpy as jnp
from jax import lax
from jax.experimental import pallas as pl
from jax.experimental.pallas import tpu as pltpu

B, S, D, DC, H, DH, DR = 4, 256, 4096, 256, 32, 128, 64
DC_SH = DC // 2
BS = B * S
SCALE = (DH + DR) ** -0.5
SCALE2 = SCALE * 1.4426950408889634
BF16 = jnp.bfloat16

_QBN = 256
_QSTEPS = D // _QBN
_HG = _QBN // DH
_LAG = 4
_GRID = _QSTEPS + _LAG
_QWIN = _LAG + 1
_NQUART = 4


_KBN = 512
_KSTEPS = D // _KBN


def _qr_cast_body(x_ref, wqr_ref, wkr_ref, wdkv_ref, wuk_ref, wuv_ref,
                  xb_ref, qr_ref, kr_ref, wdkvb_ref, wukb_ref, wuvb_ref,
                  acc_qr, acc_kr):
    j = pl.program_id(0)
    xbj = x_ref[...].reshape(BS, _KBN).astype(BF16)
    xb_ref[...] = xbj
    wdkvb_ref[...] = wdkv_ref[...].astype(BF16)
    for i in range(2):
        cols = slice(i * _QBN, (i + 1) * _QBN)
        wukb_ref[i] = wuk_ref[:, cols].astype(BF16)
        wuvb_ref[i] = wuv_ref[:, cols].astype(BF16)
    nqr = acc_qr.shape[1]
    sb = nqr // 2
    dqr = []
    for i in range(2):
        wb = wqr_ref[:, i * sb:(i + 1) * sb].astype(BF16)
        dqr.append(jnp.dot(xbj, wb, preferred_element_type=jnp.float32))
    dkr = jnp.dot(xbj, wkr_ref[...].astype(BF16),
                  preferred_element_type=jnp.float32)

    @pl.when(j == 0)
    def _():
        for i in range(2):
            acc_qr[:, i * sb:(i + 1) * sb] = dqr[i]
        acc_kr[...] = dkr

    @pl.when(j > 0)
    def _():
        for i in range(2):
            acc_qr[:, i * sb:(i + 1) * sb] += dqr[i]
        acc_kr[...] += dkr

    @pl.when(j == _KSTEPS - 1)
    def _():
        qr_ref[...] = (acc_qr[...] * SCALE2).astype(BF16)
        kr_ref[...] = acc_kr[...].astype(BF16)


def _qr_cast(x, wqr, wkr, wdkv, wuk, wuv):
    nqr = wqr.shape[1]
    nkr = wkr.shape[1]
    return pl.pallas_call(
        _qr_cast_body,
        grid=(_KSTEPS,),
        out_shape=[
            jax.ShapeDtypeStruct((BS, D), BF16),
            jax.ShapeDtypeStruct((BS, nqr), BF16),
            jax.ShapeDtypeStruct((BS, nkr), BF16),
            jax.ShapeDtypeStruct((D, DC_SH), BF16),
            jax.ShapeDtypeStruct((_QSTEPS, DC_SH, _QBN), BF16),
            jax.ShapeDtypeStruct((_QSTEPS, DC_SH, _QBN), BF16),
        ],
        in_specs=[
            pl.BlockSpec((B, S, _KBN), lambda j: (0, 0, j)),
            pl.BlockSpec((_KBN, nqr), lambda j: (j, 0)),
            pl.BlockSpec((_KBN, nkr), lambda j: (j, 0)),
            pl.BlockSpec((_KBN, DC_SH), lambda j: (j, 0)),
            pl.BlockSpec((DC_SH, 2 * _QBN), lambda j: (0, j)),
            pl.BlockSpec((DC_SH, 2 * _QBN), lambda j: (0, j)),
        ],
        out_specs=[
            pl.BlockSpec((BS, _KBN), lambda j: (0, j)),
            pl.BlockSpec((BS, nqr), lambda j: (0, 0)),
            pl.BlockSpec((BS, nkr), lambda j: (0, 0)),
            pl.BlockSpec((_KBN, DC_SH), lambda j: (j, 0)),
            pl.BlockSpec((2, DC_SH, _QBN), lambda j: (j, 0, 0)),
            pl.BlockSpec((2, DC_SH, _QBN), lambda j: (j, 0, 0)),
        ],
        scratch_shapes=[
            pltpu.VMEM((BS, nqr), jnp.float32),
            pltpu.VMEM((BS, nkr), jnp.float32),
        ],
        compiler_params=pltpu.CompilerParams(
            dimension_semantics=("arbitrary",),
        ),
    )(x, wqr, wkr, wdkv, wuk, wuv)


def _mk_rdma(src, dst, send_sems, recv_sems, slot, nbr):
    return pltpu.make_async_remote_copy(
        src_ref=src, dst_ref=dst,
        send_sem=send_sems.at[slot], recv_sem=recv_sems.at[slot],
        device_id=nbr, device_id_type=pl.DeviceIdType.MESH)


_QSZ = _QSTEPS // _NQUART


def _fused_body(xb_ref, wdkvb_ref, wukb_ref, wuvb_ref, kr_ref,
                wq_ref, qr_ref, o_ref,
                cf_ref, wkf_ref, wvf_ref, q_scr, send_sems, recv_sems):
    j = pl.program_id(0)
    my_x = lax.axis_index("x")
    my_y = lax.axis_index("y")
    my_z = lax.axis_index("z")
    nbr = (my_x, 1 - my_y, my_z)
    yslot = pl.ds(my_y, 1)

    def rdma_c():
        return _mk_rdma(cf_ref.at[yslot], cf_ref.at[yslot],
                        send_sems, recv_sems, 0, nbr)

    def rdma_w(qi):
        qs = pl.ds(qi * _QSZ, _QSZ)
        return (
            _mk_rdma(wkf_ref.at[qs, yslot], wkf_ref.at[qs, yslot],
                     send_sems, recv_sems, 1 + qi, nbr),
            _mk_rdma(wvf_ref.at[qs, yslot], wvf_ref.at[qs, yslot],
                     send_sems, recv_sems, 1 + _NQUART + qi, nbr),
        )

    @pl.when(j == 0)
    def _():
        barrier = pltpu.get_barrier_semaphore()
        pl.semaphore_signal(barrier, inc=1, device_id=nbr,
                            device_id_type=pl.DeviceIdType.MESH)
        pl.semaphore_wait(barrier, 1)

        c = jnp.dot(xb_ref[...], wdkvb_ref[...],
                    preferred_element_type=jnp.float32)
        cf_ref[yslot] = c.astype(BF16)[None]
        rdma_c().start()
        for i in range(_QSTEPS):
            wkf_ref[i, yslot] = wukb_ref[i][None]
            wvf_ref[i, yslot] = wuvb_ref[i][None]
        for qi in range(_NQUART):
            r_wk, r_wv = rdma_w(qi)
            r_wk.start()
            r_wv.start()

    def q_step():
        qblk = (jnp.dot(xb_ref[...], wq_ref[...].astype(BF16),
                        preferred_element_type=jnp.float32)
                * SCALE2).astype(BF16)
        q_scr[pl.ds(lax.rem(j, _QWIN), 1)] = qblk[None]

    def attn_step():
        hg = j - _LAG
        q_hg = q_scr[pl.ds(lax.rem(hg, _QWIN), 1)][0]
        c0 = cf_ref[0]
        c1 = cf_ref[1]
        wk_blk = wkf_ref[pl.ds(hg, 1)][0]
        wv_blk = wvf_ref[pl.ds(hg, 1)][0]
        k_hg = (jnp.dot(c0, wk_blk[0], preferred_element_type=jnp.float32)
                + jnp.dot(c1, wk_blk[1], preferred_element_type=jnp.float32)
                ).astype(BF16)
        v_hg = (jnp.dot(c0, wv_blk[0], preferred_element_type=jnp.float32)
                + jnp.dot(c1, wv_blk[1], preferred_element_type=jnp.float32)
                ).astype(BF16)
        qr = qr_ref[...]
        kr = kr_ref[...]
        dn = (((1,), (1,)), ((), ()))
        rows_out = []
        for b in range(B):
            rb = slice(b * S, (b + 1) * S)
            krb = kr[rb, :]
            outs = []
            for h in range(_HG):
                qh = q_hg[rb, h * DH:(h + 1) * DH]
                kh = k_hg[rb, h * DH:(h + 1) * DH]
                vh = v_hg[rb, h * DH:(h + 1) * DH]
                qrh = qr[rb, h * DR:(h + 1) * DR]
                s = lax.dot_general(qh, kh, dn,
                                    preferred_element_type=jnp.float32)
                s = s + lax.dot_general(qrh, krb, dn,
                                        preferred_element_type=jnp.float32)
                p = jnp.exp2(s)
                o = lax.dot_general(p.astype(BF16), vh,
                                    (((1,), (0,)), ((), ())),
                                    preferred_element_type=jnp.float32)
                o = o * (1.0 / jnp.sum(p, axis=-1, keepdims=True))
                outs.append(o.astype(BF16))
            rows_out.append(jnp.concatenate(outs, axis=1))
        o_ref[...] = jnp.concatenate(rows_out, axis=0)

    @pl.when(j == 2)
    def _():
        rdma_c().wait()

    for _qi in range(_NQUART):
        @pl.when(j == _qi * _QSZ + _LAG)
        def _(qi=_qi):
            r_wk, r_wv = rdma_w(qi)
            r_wk.wait()
            r_wv.wait()

    @pl.when(j < _LAG)
    def _():
        q_step()

    @pl.when((j >= _LAG) & (j < _QSTEPS))
    def _():
        q_step()
        attn_step()

    @pl.when(j >= _QSTEPS)
    def _():
        attn_step()


def _fused(xb, wdkvb, wukb, wuvb, kr, wq, qr):
    last = _QSTEPS - 1

    def qmap(j):
        return (0, jnp.minimum(j, last))

    def amap(j):
        return (0, jnp.maximum(j - _LAG, 0))

    return pl.pallas_call(
        _fused_body,
        grid=(_GRID,),
        out_shape=jax.ShapeDtypeStruct((BS, D), BF16),
        in_specs=[
            pl.BlockSpec((BS, D), lambda j: (0, 0)),
            pl.BlockSpec((D, DC_SH), lambda j: (0, 0)),
            pl.BlockSpec((_QSTEPS, DC_SH, _QBN), lambda j: (0, 0, 0)),
            pl.BlockSpec((_QSTEPS, DC_SH, _QBN), lambda j: (0, 0, 0)),
            pl.BlockSpec((BS, DR), lambda j: (0, 0)),
            pl.BlockSpec((D, _QBN), qmap),
            pl.BlockSpec((BS, _HG * DR), amap),
        ],
        out_specs=pl.BlockSpec((BS, _QBN), amap),
        scratch_shapes=[
            pltpu.VMEM((2, BS, DC_SH), BF16),
            pltpu.VMEM((_QSTEPS, 2, DC_SH, _QBN), BF16),
            pltpu.VMEM((_QSTEPS, 2, DC_SH, _QBN), BF16),
            pltpu.VMEM((_QWIN, BS, _QBN), BF16),
            pltpu.SemaphoreType.DMA((1 + 2 * _NQUART,)),
            pltpu.SemaphoreType.DMA((1 + 2 * _NQUART,)),
        ],
        compiler_params=pltpu.CompilerParams(
            collective_id=0,
            dimension_semantics=("arbitrary",),
        ),
    )(xb, wdkvb, wukb, wuvb, kr, wq, qr)


def _mm_body(x_ref, w_ref, o_ref, *, scale):
    x = x_ref[...]
    n = w_ref.shape[1]
    nsub = 2 if n % 256 == 0 else 1
    sb = n // nsub
    for i in range(nsub):
        wb = w_ref[:, i * sb:(i + 1) * sb].astype(BF16)
        acc = jnp.dot(x, wb, preferred_element_type=jnp.float32)
        if scale != 1.0:
            acc = acc * scale
        o_ref[:, i * sb:(i + 1) * sb] = acc.astype(o_ref.dtype)


def _matmul(xb, w, out_dtype, block_n=512, scale=1.0):
    m, k = xb.shape
    _, n = w.shape
    bn = min(block_n, n)
    return pl.pallas_call(
        functools.partial(_mm_body, scale=scale),
        grid=(n // bn,),
        in_specs=[
            pl.BlockSpec((m, k), lambda j: (0, 0)),
            pl.BlockSpec((k, bn), lambda j: (0, j)),
        ],
        out_specs=pl.BlockSpec((m, bn), lambda j: (0, j)),
        out_shape=jax.ShapeDtypeStruct((m, n), out_dtype),
    )(xb, w)


def kernel(x, Wdkv, Wuk, Wuv, Wq, Wqr, Wkr, Wo):
    xb, qr, kr, wdkvb, wukb, wuvb = _qr_cast(x, Wqr, Wkr, Wdkv, Wuk, Wuv)
    attn = _fused(xb, wdkvb, wukb, wuvb, kr, Wq, qr)
    out = _matmul(attn, Wo, jnp.float32)
    return out.reshape(B, S, D)


# device time: 180788 ns/iter; 1.0593x vs baseline; 1.0593x over previous
import jax
import jax.numpy as jnp
from jax import lax
from jax.experimental import pallas as pl
from jax.experimental.pallas import tpu as pltpu

B, S, D, DC, H, DH, DR = 4, 256, 4096, 256, 32, 128, 64
DC_SH = DC // 2
BS = B * S
SCALE = (DH + DR) ** -0.5
SCALE2 = SCALE * 1.4426950408889634
BF16 = jnp.bfloat16


_QBN = 512
_QSTEPS = D // _QBN


def _mk_rdmas(cf_ref, wkf_ref, wvf_ref, send_sems, recv_sems, my_y, nbr):
    slot = pl.ds(my_y, 1)
    mk = pltpu.make_async_remote_copy
    return (
        mk(src_ref=wkf_ref.at[slot], dst_ref=wkf_ref.at[slot],
           send_sem=send_sems.at[0], recv_sem=recv_sems.at[0],
           device_id=nbr, device_id_type=pl.DeviceIdType.MESH),
        mk(src_ref=wvf_ref.at[slot], dst_ref=wvf_ref.at[slot],
           send_sem=send_sems.at[1], recv_sem=recv_sems.at[1],
           device_id=nbr, device_id_type=pl.DeviceIdType.MESH),
        mk(src_ref=cf_ref.at[slot], dst_ref=cf_ref.at[slot],
           send_sem=send_sems.at[2], recv_sem=recv_sems.at[2],
           device_id=nbr, device_id_type=pl.DeviceIdType.MESH),
    )


def _exchange_q_body(xb_in_ref, wdkv_ref, wuk_ref, wuv_ref, wq_ref,
                     cf_ref, wkf_ref, wvf_ref, q_ref,
                     send_sems, recv_sems):
    j = pl.program_id(0)
    my_x = lax.axis_index("x")
    my_y = lax.axis_index("y")
    my_z = lax.axis_index("z")
    nbr = (my_x, 1 - my_y, my_z)

    @pl.when(j == 0)
    def _():
        barrier = pltpu.get_barrier_semaphore()
        pl.semaphore_signal(barrier, inc=1, device_id=nbr,
                            device_id_type=pl.DeviceIdType.MESH)
        pl.semaphore_wait(barrier, 1)

        slot = pl.ds(my_y, 1)
        wkf_ref[slot] = wuk_ref[...].astype(BF16)[None]
        wvf_ref[slot] = wuv_ref[...].astype(BF16)[None]
        rdma_wk, rdma_wv, rdma_c = _mk_rdmas(
            cf_ref, wkf_ref, wvf_ref, send_sems, recv_sems, my_y, nbr)
        rdma_wk.start()
        rdma_wv.start()

        c = jnp.dot(xb_in_ref[...], wdkv_ref[...].astype(BF16),
                    preferred_element_type=jnp.float32)
        cf_ref[slot] = c.astype(BF16)[None]
        rdma_c.start()

    xb = xb_in_ref[...]
    for i in range(2):
        sb = _QBN // 2
        wb = wq_ref[:, i * sb:(i + 1) * sb].astype(BF16)
        q_ref[:, i * sb:(i + 1) * sb] = (
            jnp.dot(xb, wb, preferred_element_type=jnp.float32) * SCALE2
        ).astype(BF16)

    @pl.when(j == _QSTEPS - 1)
    def _():
        rdma_wk, rdma_wv, rdma_c = _mk_rdmas(
            cf_ref, wkf_ref, wvf_ref, send_sems, recv_sems, my_y, nbr)
        rdma_wk.wait()
        rdma_wv.wait()
        rdma_c.wait()


def _exchange_q(xb, wdkv, wuk, wuv, wq):
    return pl.pallas_call(
        _exchange_q_body,
        grid=(_QSTEPS,),
        out_shape=[
            jax.ShapeDtypeStruct((2, BS, DC_SH), BF16),
            jax.ShapeDtypeStruct((2, DC_SH, D), BF16),
            jax.ShapeDtypeStruct((2, DC_SH, D), BF16),
            jax.ShapeDtypeStruct((BS, D), BF16),
        ],
        in_specs=[
            pl.BlockSpec((BS, D), lambda j: (0, 0)),
            pl.BlockSpec((D, DC_SH), lambda j: (0, 0)),
            pl.BlockSpec((DC_SH, D), lambda j: (0, 0)),
            pl.BlockSpec((DC_SH, D), lambda j: (0, 0)),
            pl.BlockSpec((D, _QBN), lambda j: (0, j)),
        ],
        out_specs=[
            pl.BlockSpec((2, BS, DC_SH), lambda j: (0, 0, 0)),
            pl.BlockSpec((2, DC_SH, D), lambda j: (0, 0, 0)),
            pl.BlockSpec((2, DC_SH, D), lambda j: (0, 0, 0)),
            pl.BlockSpec((BS, _QBN), lambda j: (0, j)),
        ],
        scratch_shapes=[
            pltpu.SemaphoreType.DMA((3,)),
            pltpu.SemaphoreType.DMA((3,)),
        ],
        compiler_params=pltpu.CompilerParams(
            collective_id=0,
            dimension_semantics=("arbitrary",),
        ),
    )(xb, wdkv, wuk, wuv, wq)


_KBN = 512
_KSTEPS = D // _KBN


def _qr_cast_body(x_ref, wqr_ref, wkr_ref, xb_ref, qr_ref, kr_ref,
                  acc_qr, acc_kr):
    j = pl.program_id(0)
    xbj = x_ref[...].reshape(BS, _KBN).astype(BF16)
    xb_ref[...] = xbj
    nqr = acc_qr.shape[1]
    sb = nqr // 2
    dqr = []
    for i in range(2):
        wb = wqr_ref[:, i * sb:(i + 1) * sb].astype(BF16)
        dqr.append(jnp.dot(xbj, wb, preferred_element_type=jnp.float32))
    dkr = jnp.dot(xbj, wkr_ref[...].astype(BF16),
                  preferred_element_type=jnp.float32)

    @pl.when(j == 0)
    def _():
        for i in range(2):
            acc_qr[:, i * sb:(i + 1) * sb] = dqr[i]
        acc_kr[...] = dkr

    @pl.when(j > 0)
    def _():
        for i in range(2):
            acc_qr[:, i * sb:(i + 1) * sb] += dqr[i]
        acc_kr[...] += dkr

    @pl.when(j == _KSTEPS - 1)
    def _():
        qr_ref[...] = (acc_qr[...] * SCALE2).astype(BF16)
        kr_ref[...] = acc_kr[...].astype(BF16)


def _qr_cast(x, wqr, wkr):
    nqr = wqr.shape[1]
    nkr = wkr.shape[1]
    return pl.pallas_call(
        _qr_cast_body,
        grid=(_KSTEPS,),
        out_shape=[
            jax.ShapeDtypeStruct((BS, D), BF16),
            jax.ShapeDtypeStruct((BS, nqr), BF16),
            jax.ShapeDtypeStruct((BS, nkr), BF16),
        ],
        in_specs=[
            pl.BlockSpec((B, S, _KBN), lambda j: (0, 0, j)),
            pl.BlockSpec((_KBN, nqr), lambda j: (j, 0)),
            pl.BlockSpec((_KBN, nkr), lambda j: (j, 0)),
        ],
        out_specs=[
            pl.BlockSpec((BS, _KBN), lambda j: (0, j)),
            pl.BlockSpec((BS, nqr), lambda j: (0, 0)),
            pl.BlockSpec((BS, nkr), lambda j: (0, 0)),
        ],
        scratch_shapes=[
            pltpu.VMEM((BS, nqr), jnp.float32),
            pltpu.VMEM((BS, nkr), jnp.float32),
        ],
        compiler_params=pltpu.CompilerParams(
            dimension_semantics=("arbitrary",),
        ),
    )(x, wqr, wkr)


def _mm_body(x_ref, w_ref, o_ref, *, scale):
    x = x_ref[...]
    n = w_ref.shape[1]
    nsub = 2 if n % 256 == 0 else 1
    sb = n // nsub
    for i in range(nsub):
        wb = w_ref[:, i * sb:(i + 1) * sb].astype(BF16)
        acc = jnp.dot(x, wb, preferred_element_type=jnp.float32)
        if scale != 1.0:
            acc = acc * scale
        o_ref[:, i * sb:(i + 1) * sb] = acc.astype(o_ref.dtype)


def _matmul(xb, w, out_dtype, block_n=512, scale=1.0):
    import functools
    m, k = xb.shape
    _, n = w.shape
    bn = min(block_n, n)
    return pl.pallas_call(
        functools.partial(_mm_body, scale=scale),
        grid=(n // bn,),
        in_specs=[
            pl.BlockSpec((m, k), lambda j: (0, 0)),
            pl.BlockSpec((k, bn), lambda j: (0, j)),
        ],
        out_specs=pl.BlockSpec((m, bn), lambda j: (0, j)),
        out_shape=jax.ShapeDtypeStruct((m, n), out_dtype),
    )(xb, w)


def _attn_body(cf_ref, wkf_ref, wvf_ref, q_ref, qr_ref, kr_ref, o_ref):
    b = pl.program_id(0)
    q = q_ref[0]
    qr = qr_ref[0]
    kr = kr_ref[0]
    rows = pl.ds(b * S, S)
    c0 = cf_ref[0, rows, :]
    c1 = cf_ref[1, rows, :]
    k = (jnp.dot(c0, wkf_ref[0], preferred_element_type=jnp.float32)
         + jnp.dot(c1, wkf_ref[1], preferred_element_type=jnp.float32)
         ).astype(BF16)
    v = (jnp.dot(c0, wvf_ref[0], preferred_element_type=jnp.float32)
         + jnp.dot(c1, wvf_ref[1], preferred_element_type=jnp.float32)
         ).astype(BF16)
    dn = (((1,), (1,)), ((), ()))
    outs = []
    for h in range(H):
        qh = q[:, h * DH:(h + 1) * DH]
        kh = k[:, h * DH:(h + 1) * DH]
        vh = v[:, h * DH:(h + 1) * DH]
        qrh = qr[:, h * DR:(h + 1) * DR]
        s = lax.dot_general(qh, kh, dn, preferred_element_type=jnp.float32)
        s = s + lax.dot_general(qrh, kr, dn,
                                preferred_element_type=jnp.float32)
        p = jnp.exp2(s)
        o = lax.dot_general(p.astype(BF16), vh, (((1,), (0,)), ((), ())),
                            preferred_element_type=jnp.float32)
        o = o * (1.0 / jnp.sum(p, axis=-1, keepdims=True))
        outs.append(o.astype(BF16))
    o_ref[0] = jnp.concatenate(outs, axis=1)


def _attention(cf, wkf, wvf, q, qr, kr):
    q = q.reshape(B, S, H * DH)
    qr = qr.reshape(B, S, H * DR)
    kr = kr.reshape(B, S, DR)
    return pl.pallas_call(
        _attn_body,
        grid=(B,),
        in_specs=[
            pl.BlockSpec((2, BS, DC_SH), lambda b: (0, 0, 0)),
            pl.BlockSpec((2, DC_SH, D), lambda b: (0, 0, 0)),
            pl.BlockSpec((2, DC_SH, D), lambda b: (0, 0, 0)),
            pl.BlockSpec((1, S, H * DH), lambda b: (b, 0, 0)),
            pl.BlockSpec((1, S, H * DR), lambda b: (b, 0, 0)),
            pl.BlockSpec((1, S, DR), lambda b: (b, 0, 0)),
        ],
        out_specs=pl.BlockSpec((1, S, H * DH), lambda b: (b, 0, 0)),
        out_shape=jax.ShapeDtypeStruct((B, S, H * DH), BF16),
        compiler_params=pltpu.CompilerParams(
            dimension_semantics=("arbitrary",),
        ),
    )(cf, wkf, wvf, q, qr, kr)


def kernel(x, Wdkv, Wuk, Wuv, Wq, Wqr, Wkr, Wo):
    xb, qr, kr = _qr_cast(x, Wqr, Wkr)
    cf, wkf, wvf, q = _exchange_q(xb, Wdkv, Wuk, Wuv, Wq)
    attn = _attention(cf, wkf, wvf, q, qr, kr)
    out = _matmul(attn.reshape(BS, D), Wo, jnp.float32)
    return out.reshape(B, S, D)


# device time: 180133 ns/iter; 1.0631x vs baseline; 1.0036x over previous
import jax
import jax.numpy as jnp
from jax import lax
from jax.experimental import pallas as pl
from jax.experimental.pallas import tpu as pltpu

B, S, D, DC, H, DH, DR = 4, 256, 4096, 256, 32, 128, 64
DC_SH = DC // 2
BS = B * S
SCALE = (DH + DR) ** -0.5
SCALE2 = SCALE * 1.4426950408889634
BF16 = jnp.bfloat16


_QBN = 512
_QSTEPS = D // _QBN


def _mk_rdmas(cf_ref, wkf_ref, wvf_ref, send_sems, recv_sems, my_y, nbr):
    slot = pl.ds(my_y, 1)
    mk = pltpu.make_async_remote_copy
    return (
        mk(src_ref=wkf_ref.at[slot], dst_ref=wkf_ref.at[slot],
           send_sem=send_sems.at[0], recv_sem=recv_sems.at[0],
           device_id=nbr, device_id_type=pl.DeviceIdType.MESH),
        mk(src_ref=wvf_ref.at[slot], dst_ref=wvf_ref.at[slot],
           send_sem=send_sems.at[1], recv_sem=recv_sems.at[1],
           device_id=nbr, device_id_type=pl.DeviceIdType.MESH),
        mk(src_ref=cf_ref.at[slot], dst_ref=cf_ref.at[slot],
           send_sem=send_sems.at[2], recv_sem=recv_sems.at[2],
           device_id=nbr, device_id_type=pl.DeviceIdType.MESH),
    )


def _exchange_q_body(xb_in_ref, wdkv_ref, wuk_ref, wuv_ref, wq_ref,
                     cf_ref, wkf_ref, wvf_ref, q_ref,
                     send_sems, recv_sems):
    j = pl.program_id(0)
    my_x = lax.axis_index("x")
    my_y = lax.axis_index("y")
    my_z = lax.axis_index("z")
    nbr = (my_x, 1 - my_y, my_z)

    @pl.when(j == 0)
    def _():
        barrier = pltpu.get_barrier_semaphore()
        pl.semaphore_signal(barrier, inc=1, device_id=nbr,
                            device_id_type=pl.DeviceIdType.MESH)
        pl.semaphore_wait(barrier, 1)

        slot = pl.ds(my_y, 1)
        wkf_ref[slot] = wuk_ref[...].astype(BF16)[None]
        wvf_ref[slot] = wuv_ref[...].astype(BF16)[None]
        rdma_wk, rdma_wv, rdma_c = _mk_rdmas(
            cf_ref, wkf_ref, wvf_ref, send_sems, recv_sems, my_y, nbr)
        rdma_wk.start()
        rdma_wv.start()

        c = jnp.dot(xb_in_ref[...], wdkv_ref[...].astype(BF16),
                    preferred_element_type=jnp.float32)
        cf_ref[slot] = c.astype(BF16)[None]
        rdma_c.start()

    xb = xb_in_ref[...]
    for i in range(2):
        sb = _QBN // 2
        wb = wq_ref[:, i * sb:(i + 1) * sb].astype(BF16)
        q_ref[:, i * sb:(i + 1) * sb] = (
            jnp.dot(xb, wb, preferred_element_type=jnp.float32) * SCALE2
        ).astype(BF16)

    @pl.when(j == _QSTEPS - 1)
    def _():
        rdma_wk, rdma_wv, rdma_c = _mk_rdmas(
            cf_ref, wkf_ref, wvf_ref, send_sems, recv_sems, my_y, nbr)
        rdma_wk.wait()
        rdma_wv.wait()
        rdma_c.wait()


def _exchange_q(xb, wdkv, wuk, wuv, wq):
    return pl.pallas_call(
        _exchange_q_body,
        grid=(_QSTEPS,),
        out_shape=[
            jax.ShapeDtypeStruct((2, BS, DC_SH), BF16),
            jax.ShapeDtypeStruct((2, DC_SH, D), BF16),
            jax.ShapeDtypeStruct((2, DC_SH, D), BF16),
            jax.ShapeDtypeStruct((BS, D), BF16),
        ],
        in_specs=[
            pl.BlockSpec((BS, D), lambda j: (0, 0)),
            pl.BlockSpec((D, DC_SH), lambda j: (0, 0)),
            pl.BlockSpec((DC_SH, D), lambda j: (0, 0)),
            pl.BlockSpec((DC_SH, D), lambda j: (0, 0)),
            pl.BlockSpec((D, _QBN), lambda j: (0, j)),
        ],
        out_specs=[
            pl.BlockSpec((2, BS, DC_SH), lambda j: (0, 0, 0)),
            pl.BlockSpec((2, DC_SH, D), lambda j: (0, 0, 0)),
            pl.BlockSpec((2, DC_SH, D), lambda j: (0, 0, 0)),
            pl.BlockSpec((BS, _QBN), lambda j: (0, j)),
        ],
        scratch_shapes=[
            pltpu.SemaphoreType.DMA((3,)),
            pltpu.SemaphoreType.DMA((3,)),
        ],
        compiler_params=pltpu.CompilerParams(
            collective_id=0,
            dimension_semantics=("arbitrary",),
        ),
    )(xb, wdkv, wuk, wuv, wq)


_KBN = 512
_KSTEPS = D // _KBN


def _qr_cast_body(x_ref, wqr_ref, wkr_ref, xb_ref, qr_ref, kr_ref,
                  acc_qr, acc_kr):
    j = pl.program_id(0)
    xbj = x_ref[...].reshape(BS, _KBN).astype(BF16)
    xb_ref[...] = xbj
    nqr = acc_qr.shape[1]
    sb = nqr // 2
    dqr = []
    for i in range(2):
        wb = wqr_ref[:, i * sb:(i + 1) * sb].astype(BF16)
        dqr.append(jnp.dot(xbj, wb, preferred_element_type=jnp.float32))
    dkr = jnp.dot(xbj, wkr_ref[...].astype(BF16),
                  preferred_element_type=jnp.float32)

    @pl.when(j == 0)
    def _():
        for i in range(2):
            acc_qr[:, i * sb:(i + 1) * sb] = dqr[i]
        acc_kr[...] = dkr

    @pl.when(j > 0)
    def _():
        for i in range(2):
            acc_qr[:, i * sb:(i + 1) * sb] += dqr[i]
        acc_kr[...] += dkr

    @pl.when(j == _KSTEPS - 1)
    def _():
        qr_ref[...] = (acc_qr[...] * SCALE2).astype(BF16)
        kr_ref[...] = acc_kr[...].astype(BF16)


def _qr_cast(x, wqr, wkr):
    nqr = wqr.shape[1]
    nkr = wkr.shape[1]
    return pl.pallas_call(
        _qr_cast_body,
        grid=(_KSTEPS,),
        out_shape=[
            jax.ShapeDtypeStruct((BS, D), BF16),
            jax.ShapeDtypeStruct((BS, nqr), BF16),
            jax.ShapeDtypeStruct((BS, nkr), BF16),
        ],
        in_specs=[
            pl.BlockSpec((B, S, _KBN), lambda j: (0, 0, j)),
            pl.BlockSpec((_KBN, nqr), lambda j: (j, 0)),
            pl.BlockSpec((_KBN, nkr), lambda j: (j, 0)),
        ],
        out_specs=[
            pl.BlockSpec((BS, _KBN), lambda j: (0, j)),
            pl.BlockSpec((BS, nqr), lambda j: (0, 0)),
            pl.BlockSpec((BS, nkr), lambda j: (0, 0)),
        ],
        scratch_shapes=[
            pltpu.VMEM((BS, nqr), jnp.float32),
            pltpu.VMEM((BS, nkr), jnp.float32),
        ],
        compiler_params=pltpu.CompilerParams(
            dimension_semantics=("arbitrary",),
        ),
    )(x, wqr, wkr)


def _mm_body(x_ref, w_ref, o_ref, *, scale):
    x = x_ref[...]
    n = w_ref.shape[1]
    nsub = 2 if n % 256 == 0 else 1
    sb = n // nsub
    for i in range(nsub):
        wb = w_ref[:, i * sb:(i + 1) * sb].astype(BF16)
        acc = jnp.dot(x, wb, preferred_element_type=jnp.float32)
        if scale != 1.0:
            acc = acc * scale
        o_ref[:, i * sb:(i + 1) * sb] = acc.astype(o_ref.dtype)


def _matmul(xb, w, out_dtype, block_n=512, scale=1.0):
    import functools
    m, k = xb.shape
    _, n = w.shape
    bn = min(block_n, n)
    return pl.pallas_call(
        functools.partial(_mm_body, scale=scale),
        grid=(n // bn,),
        in_specs=[
            pl.BlockSpec((m, k), lambda j: (0, 0)),
            pl.BlockSpec((k, bn), lambda j: (0, j)),
        ],
        out_specs=pl.BlockSpec((m, bn), lambda j: (0, j)),
        out_shape=jax.ShapeDtypeStruct((m, n), out_dtype),
    )(xb, w)


def _attn_body(cf_ref, wkf_ref, wvf_ref, q_ref, qr_ref, kr_ref, o_ref):
    b = pl.program_id(0)
    q = q_ref[...]
    qr = qr_ref[...]
    kr = kr_ref[...]
    rows = pl.ds(b * S, S)
    c0 = cf_ref[0, rows, :]
    c1 = cf_ref[1, rows, :]
    k = (jnp.dot(c0, wkf_ref[0], preferred_element_type=jnp.float32)
         + jnp.dot(c1, wkf_ref[1], preferred_element_type=jnp.float32)
         ).astype(BF16)
    v = (jnp.dot(c0, wvf_ref[0], preferred_element_type=jnp.float32)
         + jnp.dot(c1, wvf_ref[1], preferred_element_type=jnp.float32)
         ).astype(BF16)
    dn = (((1,), (1,)), ((), ()))
    outs = []
    for h in range(H):
        qh = q[:, h * DH:(h + 1) * DH]
        kh = k[:, h * DH:(h + 1) * DH]
        vh = v[:, h * DH:(h + 1) * DH]
        qrh = qr[:, h * DR:(h + 1) * DR]
        s = lax.dot_general(qh, kh, dn, preferred_element_type=jnp.float32)
        s = s + lax.dot_general(qrh, kr, dn,
                                preferred_element_type=jnp.float32)
        p = jnp.exp2(s)
        o = lax.dot_general(p.astype(BF16), vh, (((1,), (0,)), ((), ())),
                            preferred_element_type=jnp.float32)
        o = o * (1.0 / jnp.sum(p, axis=-1, keepdims=True))
        outs.append(o.astype(BF16))
    o_ref[...] = jnp.concatenate(outs, axis=1)


def _attention(cf, wkf, wvf, q, qr, kr):
    return pl.pallas_call(
        _attn_body,
        grid=(B,),
        in_specs=[
            pl.BlockSpec((2, BS, DC_SH), lambda b: (0, 0, 0)),
            pl.BlockSpec((2, DC_SH, D), lambda b: (0, 0, 0)),
            pl.BlockSpec((2, DC_SH, D), lambda b: (0, 0, 0)),
            pl.BlockSpec((S, H * DH), lambda b: (b, 0)),
            pl.BlockSpec((S, H * DR), lambda b: (b, 0)),
            pl.BlockSpec((S, DR), lambda b: (b, 0)),
        ],
        out_specs=pl.BlockSpec((S, H * DH), lambda b: (b, 0)),
        out_shape=jax.ShapeDtypeStruct((BS, H * DH), BF16),
        compiler_params=pltpu.CompilerParams(
            dimension_semantics=("arbitrary",),
        ),
    )(cf, wkf, wvf, q, qr, kr)


def kernel(x, Wdkv, Wuk, Wuv, Wq, Wqr, Wkr, Wo):
    xb, qr, kr = _qr_cast(x, Wqr, Wkr)
    cf, wkf, wvf, q = _exchange_q(xb, Wdkv, Wuk, Wuv, Wq)
    attn = _attention(cf, wkf, wvf, q, qr, kr)
    out = _matmul(attn, Wo, jnp.float32)
    return out.reshape(B, S, D)
